# Initial kernel scaffold; baseline (speedup 1.0000x reference)
import jax
import jax.numpy as jnp
from jax.experimental import pallas as pl


def kernel(index, length, data):
    raise NotImplementedError("write your pallas kernel here")


# speedup vs baseline: 4.2619x; 4.2619x over previous
import functools

import jax
import jax.numpy as jnp
from jax import lax
from jax.experimental import pallas as pl
from jax.experimental.pallas import tpu as pltpu
from jax.experimental.pallas import tpu_sc as plsc

_NC = 2
_NS = 16
_NW = _NC * _NS
_WIN = 200
_NBUF = 8
_LAG = 4


def _sc_window_gather(starts, data_flat, wrap_flat, C, D):
    B = starts.shape[0]
    per_w = B // _NW
    win = _WIN * D

    mesh = plsc.VectorSubcoreMesh(
        core_axis_name="c", subcore_axis_name="s",
        num_cores=_NC, num_subcores=_NS)

    @functools.partial(
        pl.kernel,
        out_type=jax.ShapeDtypeStruct((B * win,), jnp.float32),
        mesh=mesh,
        scratch_types=[
            pltpu.VMEM((per_w,), jnp.int32),
            [pltpu.VMEM((win,), jnp.float32) for _ in range(_NBUF)],
            pltpu.SemaphoreType.DMA((_NBUF,)),
            pltpu.SemaphoreType.DMA((_NBUF,)),
        ],
    )
    def run(starts_hbm, tab_hbm, wrap_hbm, out_hbm, idx_v, bufs, in_sem, out_sem):
        wid = lax.axis_index("c") * _NS + lax.axis_index("s")
        base = wid * per_w
        pltpu.sync_copy(starts_hbm.at[pl.ds(base, per_w)], idx_v)

        in_d = [None] * per_w
        out_d = [None] * per_w

        def issue_out(f):
            fslot = f % _NBUF
            in_d[f].wait()
            d = pltpu.make_async_copy(
                bufs[fslot],
                out_hbm.at[pl.ds((base + f) * win, win)],
                out_sem.at[fslot])
            d.start()
            out_d[f] = d

        for o in range(per_w // 16):
            v = idx_v[pl.ds(o * 16, 16)]
            v = jnp.where(v >= C, v - C, v)
            for l in range(16):
                e = o * 16 + l
                slot = e % _NBUF
                if e >= _NBUF:
                    out_d[e - _NBUF].wait()
                s = v[l]
                wrapped = s > C - _WIN

                @pl.when(wrapped)
                def _():
                    pltpu.make_async_copy(
                        wrap_hbm.at[pl.ds((s - (C - _WIN)) * D, win)],
                        bufs[slot], in_sem.at[slot]).start()

                @pl.when(jnp.logical_not(wrapped))
                def _():
                    pltpu.make_async_copy(
                        tab_hbm.at[pl.ds(s * D, win)],
                        bufs[slot], in_sem.at[slot]).start()

                in_d[e] = pltpu.make_async_copy(
                    tab_hbm.at[pl.ds(0, win)], bufs[slot],
                    in_sem.at[slot])
                if e >= _LAG:
                    issue_out(e - _LAG)

        for f in range(per_w - _LAG, per_w):
            issue_out(f)
        for f in range(per_w - _NBUF, per_w):
            out_d[f].wait()

    return run(starts, data_flat, wrap_flat)


def kernel(index, length, data):
    C, D = data.shape
    B = index.shape[0]
    shift = jnp.mod(jnp.asarray(length, jnp.int32) - _WIN, C)
    starts = index.astype(jnp.int32) + shift
    wrap_tab = jnp.concatenate([data[C - _WIN:], data[:_WIN]], axis=0)
    out = _sc_window_gather(starts, data.reshape(-1), wrap_tab.reshape(-1), C, D)
    return out.reshape(B, _WIN, D)


# speedup vs baseline: 5.7548x; 1.3503x over previous
import functools

import jax
import jax.numpy as jnp
from jax import lax
from jax.experimental import pallas as pl
from jax.experimental.pallas import tpu as pltpu
from jax.experimental.pallas import tpu_sc as plsc

_NC = 2
_NS = 16
_NW = _NC * _NS
_WIN = 200
_FETCH = 208
_AUX = 512
_NBUF = 4
_LAG = 2
_GRP = 16


def _sc_window_gather(idx32, shift16, data, aux):
    B = idx32.shape[0]
    C, D = data.shape
    per_w = B // _NW

    mesh = plsc.VectorSubcoreMesh(
        core_axis_name="c", subcore_axis_name="s",
        num_cores=_NC, num_subcores=_NS)

    @functools.partial(
        pl.kernel,
        out_type=jax.ShapeDtypeStruct((B * _WIN, D), jnp.float32),
        mesh=mesh,
        scratch_types=[
            pltpu.VMEM((per_w,), jnp.int32),
            pltpu.VMEM((16,), jnp.int32),
            pltpu.SMEM((_NBUF,), jnp.int32),
            [pltpu.VMEM((_FETCH, D), jnp.float32) for _ in range(_NBUF)],
            [pltpu.SemaphoreType.DMA for _ in range(2 * _NBUF)],
        ],
    )
    def run(idx_hbm, shift_hbm, tab_hbm, aux_hbm, out_hbm, idx_v, shift_v,
            r0_s, bufs, sems):
        wid = lax.axis_index("c") * _NS + lax.axis_index("s")
        base = wid * per_w
        pltpu.sync_copy(idx_hbm.at[pl.ds(base, per_w)], idx_v)
        pltpu.sync_copy(shift_hbm, shift_v)
        shift = shift_v[pl.ds(0, 16)][0]

        def wait_in(j):
            pltpu.make_async_copy(
                tab_hbm.at[pl.ds(0, _FETCH)], bufs[j], sems[j]).wait()

        def start_out(f, j):
            pltpu.make_async_copy(
                bufs[j].at[pl.ds(r0_s[j], _WIN)],
                out_hbm.at[pl.ds((base + f) * _WIN, _WIN)],
                sems[_NBUF + j]).start()

        def wait_out(j):
            pltpu.make_async_copy(
                bufs[j].at[pl.ds(0, _WIN)], out_hbm.at[pl.ds(0, _WIN)],
                sems[_NBUF + j]).wait()

        def group_body(g, carry):
            v16 = idx_v[pl.ds(g * _GRP, _GRP)]
            for l in range(_GRP):
                e = g * _GRP + l
                j = l % _NBUF

                if l >= _NBUF:
                    wait_out(j)
                else:

                    @pl.when(g > 0)
                    def _():
                        wait_out(j)

                s = v16[l] + shift
                s = jnp.where(s >= C, s - C, s)
                r0 = jnp.bitwise_and(s, 7)
                r0_s[j] = r0
                a = s - r0
                near_end = s > C - _FETCH

                @pl.when(near_end)
                def _():
                    pltpu.make_async_copy(
                        aux_hbm.at[pl.ds(
                            pl.multiple_of(a - (C - _AUX // 2), 8), _FETCH)],
                        bufs[j], sems[j]).start()

                @pl.when(jnp.logical_not(near_end))
                def _():
                    pltpu.make_async_copy(
                        tab_hbm.at[pl.ds(pl.multiple_of(a, 8), _FETCH)],
                        bufs[j], sems[j]).start()

                f = e - _LAG
                fj = (l - _LAG) % _NBUF
                if l >= _LAG:
                    wait_in(fj)
                    start_out(f, fj)
                else:

                    @pl.when(g > 0)
                    def _():
                        wait_in(fj)
                        start_out(f, fj)
            return carry

        lax.fori_loop(0, per_w // _GRP, group_body, jnp.int32(0))

        for r in range(_LAG):
            f = per_w - _LAG + r
            fj = f % _NBUF
            wait_in(fj)
            start_out(jnp.int32(f), fj)
        for j in range(_NBUF):
            wait_out(j)

    return run(idx32, shift16, data, aux)


def kernel(index, length, data):
    C, D = data.shape
    B = index.shape[0]
    idx32 = index.astype(jnp.int32)
    shift = jnp.mod(jnp.asarray(length, jnp.int32) - _WIN, C)
    shift16 = jnp.full((16,), shift, jnp.int32)
    aux = jnp.concatenate([data[C - _AUX // 2:], data[:_AUX // 2]], axis=0)
    out = _sc_window_gather(idx32, shift16, data, aux)
    return out.reshape(B, _WIN, D)
